# baseline (speedup 1.0000x reference)
import numpy as np
import jax
import jax.numpy as jnp
from jax.experimental import pallas as pl
from jax.experimental.pallas import tpu as pltpu


def _band_sel(k, cin, cout, n_px, lane_fn):
    rows = k * k * cin
    sel = np.zeros((6, 96, 4, n_px, rows + 1), np.float32)
    sel[..., rows] = 1.0
    for qy in range(2):
        for qx in range(2):
            q = qy * 2 + qx
            for dy in range(k):
                for dx in range(k):
                    for ci in range(cin):
                        for px in range(n_px):
                            i = qy + dy
                            l = lane_fn(px, qx + dx, ci)
                            r = dy * k * cin + dx * cin + ci
                            sel[i, l, q, px, rows] = 0.0
                            sel[i, l, q, px, r] = 1.0
    return sel.reshape(-1, rows + 1)


_W1_SEL = _band_sel(5, 3, 6, 14, lambda px, d, ci: 32 * ci + 2 * px + d)
_W2_SEL = _band_sel(5, 6, 16, 5, lambda px, d, ci: 6 * (2 * px + d) + ci)


def _banded(w_flat, sel, n_px, cout):
    cols = jnp.concatenate([w_flat[:, :cout],
                            jnp.zeros((1, cout), w_flat.dtype)]).astype(jnp.bfloat16)
    band = jnp.dot(sel, cols, preferred_element_type=jnp.float32)
    band = band.reshape(6, 96, 4, n_px * cout)
    band = jnp.pad(band, ((0, 0), (0, 0), (0, 0), (0, 96 - n_px * cout)))
    return band.reshape(6, 96, 384).astype(jnp.bfloat16)


def _pool_bias_relu(h, b):
    m = jnp.maximum(jnp.maximum(h[:, 0:96], h[:, 96:192]),
                    jnp.maximum(h[:, 192:288], h[:, 288:384]))
    return jnp.maximum(m + b, 0.0)


def _net_kernel(x_ref, w1_ref, b1_ref, w2_ref, b2_ref,
                f1_ref, f1b_ref, f2_ref, f2b_ref, f3_ref, f3b_ref, o_ref):
    bb = x_ref.shape[1]
    x = x_ref[...]

    h = None
    for i in range(6):
        s = (16 if i % 2 else 0) + i // 2
        a = x[s:s + 14].reshape(14 * bb, 96)
        d = jnp.dot(a, w1_ref[i], preferred_element_type=jnp.float32)
        h = d if h is None else h + d
    y1 = _pool_bias_relu(h, b1_ref[...]).astype(jnp.bfloat16).reshape(14, bb, 96)
    y1 = jnp.concatenate([y1[2 * p:2 * p + 1] for p in range(7)] +
                         [y1[2 * p + 1:2 * p + 2] for p in range(7)], axis=0)

    h = None
    for i in range(6):
        s = (7 if i % 2 else 0) + i // 2
        a = y1[s:s + 5].reshape(5 * bb, 96)
        d = jnp.dot(a, w2_ref[i], preferred_element_type=jnp.float32)
        h = d if h is None else h + d
    y2 = _pool_bias_relu(h, b2_ref[...]).astype(jnp.bfloat16).reshape(5, bb, 96)

    h = None
    for r in range(5):
        d = jnp.dot(y2[r], f1_ref[r], preferred_element_type=jnp.float32)
        h = d if h is None else h + d
    h = jnp.maximum(h + f1b_ref[...], 0.0).astype(jnp.bfloat16)
    h = jnp.dot(h, f2_ref[...], preferred_element_type=jnp.float32)
    h = jnp.maximum(h + f2b_ref[...], 0.0).astype(jnp.bfloat16)
    h = jnp.dot(h, f3_ref[...], preferred_element_type=jnp.float32)
    o_ref[...] = h + f3b_ref[...]


_ROW_PERM = np.concatenate([np.arange(0, 32, 2), np.arange(1, 32, 2)])


def kernel(x, conv1_w, conv1_b, conv2_w, conv2_b, fc1_w, fc1_b,
           fc2_w, fc2_b, fc3_w, fc3_b):
    B = x.shape[0]
    bb = 128 if B % 128 == 0 else B
    grid = B // bb

    xro = jnp.transpose(x, (2, 0, 1, 3))[_ROW_PERM].reshape(32, B, 96)
    xro = xro.astype(jnp.bfloat16)

    w1b = _banded(conv1_w, jnp.asarray(_W1_SEL, jnp.bfloat16), 14, 6)
    w2b = _banded(conv2_w, jnp.asarray(_W2_SEL, jnp.bfloat16), 5, 16)
    b1p = jnp.tile(conv1_b[:, :6], (1, 16))
    b2p = jnp.tile(conv2_b[:, :16], (1, 6))
    f1w = jnp.pad(fc1_w[:400].reshape(5, 80, 128),
                  ((0, 0), (0, 16), (0, 0))).astype(jnp.bfloat16)

    out = pl.pallas_call(
        _net_kernel,
        grid=(grid,),
        out_shape=jax.ShapeDtypeStruct((B, 128), jnp.float32),
        in_specs=[
            pl.BlockSpec((32, bb, 96), lambda g: (0, g, 0)),
            pl.BlockSpec((6, 96, 384), lambda g: (0, 0, 0)),
            pl.BlockSpec((1, 96), lambda g: (0, 0)),
            pl.BlockSpec((6, 96, 384), lambda g: (0, 0, 0)),
            pl.BlockSpec((1, 96), lambda g: (0, 0)),
            pl.BlockSpec((5, 96, 128), lambda g: (0, 0, 0)),
            pl.BlockSpec((1, 128), lambda g: (0, 0)),
            pl.BlockSpec((128, 128), lambda g: (0, 0)),
            pl.BlockSpec((1, 128), lambda g: (0, 0)),
            pl.BlockSpec((128, 128), lambda g: (0, 0)),
            pl.BlockSpec((1, 128), lambda g: (0, 0)),
        ],
        out_specs=pl.BlockSpec((bb, 128), lambda g: (g, 0)),
        compiler_params=pltpu.CompilerParams(
            dimension_semantics=("parallel",)),
    )(xro, w1b, b1p, w2b, b2p, f1w, fc1_b,
      fc2_w.astype(jnp.bfloat16), fc2_b, fc3_w.astype(jnp.bfloat16), fc3_b)
    return out[:, :2]


# speedup vs baseline: 1.1855x; 1.1855x over previous
import numpy as np
import jax
import jax.numpy as jnp
from jax.experimental import pallas as pl
from jax.experimental.pallas import tpu as pltpu


def _band_sel(k, cin, cout, n_px, lane_fn):
    rows = k * k * cin
    sel = np.zeros((6, 128, 4, n_px, rows + 1), np.float32)
    sel[..., rows] = 1.0
    for qy in range(2):
        for qx in range(2):
            q = qy * 2 + qx
            for dy in range(k):
                for dx in range(k):
                    for ci in range(cin):
                        for px in range(n_px):
                            i = qy + dy
                            l = lane_fn(px, qx + dx, ci)
                            r = dy * k * cin + dx * cin + ci
                            sel[i, l, q, px, rows] = 0.0
                            sel[i, l, q, px, r] = 1.0
    return sel.reshape(-1, rows + 1)


_W1_SEL = _band_sel(5, 3, 6, 14, lambda px, d, ci: 32 * ci + 2 * px + d)
_W2_SEL = _band_sel(5, 6, 16, 5, lambda px, d, ci: 6 * (2 * px + d) + ci)


def _banded(w_flat, sel, n_px, cout):
    cols = jnp.concatenate([w_flat[:, :cout],
                            jnp.zeros((1, cout), w_flat.dtype)]).astype(jnp.bfloat16)
    band = jnp.dot(sel, cols, preferred_element_type=jnp.float32)
    band = band.reshape(6, 128, 4, n_px * cout)
    band = jnp.pad(band, ((0, 0), (0, 0), (0, 0), (0, 128 - n_px * cout)))
    return band.reshape(768, 512).astype(jnp.bfloat16)


def _pool_bias_relu(h, b):
    m = jnp.maximum(jnp.maximum(h[:, 0:128], h[:, 128:256]),
                    jnp.maximum(h[:, 256:384], h[:, 384:512]))
    return jnp.maximum(m + b, 0.0)


def _net_kernel(x_ref, w1_ref, b1_ref, w2_ref, b2_ref,
                f1_ref, f1b_ref, f2_ref, f2b_ref, f3_ref, f3b_ref, o_ref):
    bb = x_ref.shape[1]
    x = x_ref[...]

    starts = [(16 if i % 2 else 0) + i // 2 for i in range(6)]
    a = jnp.concatenate([x[s:s + 14] for s in starts], axis=2)
    h = jnp.dot(a.reshape(14 * bb, 768), w1_ref[...],
                preferred_element_type=jnp.float32)
    y1 = _pool_bias_relu(h, b1_ref[...]).astype(jnp.bfloat16).reshape(14, bb, 128)
    y1 = jnp.concatenate([y1[2 * p:2 * p + 1] for p in range(7)] +
                         [y1[2 * p + 1:2 * p + 2] for p in range(7)], axis=0)

    starts = [(7 if i % 2 else 0) + i // 2 for i in range(6)]
    a = jnp.concatenate([y1[s:s + 5] for s in starts], axis=2)
    h = jnp.dot(a.reshape(5 * bb, 768), w2_ref[...],
                preferred_element_type=jnp.float32)
    y2 = _pool_bias_relu(h, b2_ref[...]).astype(jnp.bfloat16).reshape(5, bb, 128)

    h = None
    for r in range(5):
        d = jnp.dot(y2[r], f1_ref[r], preferred_element_type=jnp.float32)
        h = d if h is None else h + d
    h = jnp.maximum(h + f1b_ref[...], 0.0).astype(jnp.bfloat16)
    h = jnp.dot(h, f2_ref[...], preferred_element_type=jnp.float32)
    h = jnp.maximum(h + f2b_ref[...], 0.0).astype(jnp.bfloat16)
    h = jnp.dot(h, f3_ref[...], preferred_element_type=jnp.float32)
    o_ref[...] = h + f3b_ref[...]


_ROW_PERM = np.concatenate([np.arange(0, 32, 2), np.arange(1, 32, 2)])


def kernel(x, conv1_w, conv1_b, conv2_w, conv2_b, fc1_w, fc1_b,
           fc2_w, fc2_b, fc3_w, fc3_b):
    B = x.shape[0]
    bb = 256 if B % 256 == 0 else B
    grid = B // bb

    xro = jnp.transpose(x, (2, 0, 1, 3))[_ROW_PERM].reshape(32, B, 96)
    xro = jnp.pad(xro, ((0, 0), (0, 0), (0, 32))).astype(jnp.bfloat16)

    w1b = _banded(conv1_w, jnp.asarray(_W1_SEL, jnp.bfloat16), 14, 6)
    w2b = _banded(conv2_w, jnp.asarray(_W2_SEL, jnp.bfloat16), 5, 16)
    b1p = jnp.pad(jnp.tile(conv1_b[:, :6], (1, 16)), ((0, 0), (0, 32)))
    b2p = jnp.tile(conv2_b[:, :16], (1, 8))
    f1w = jnp.pad(fc1_w[:400].reshape(5, 80, 128),
                  ((0, 0), (0, 48), (0, 0))).astype(jnp.bfloat16)

    out = pl.pallas_call(
        _net_kernel,
        grid=(grid,),
        out_shape=jax.ShapeDtypeStruct((B, 128), jnp.float32),
        in_specs=[
            pl.BlockSpec((32, bb, 128), lambda g: (0, g, 0)),
            pl.BlockSpec((768, 512), lambda g: (0, 0)),
            pl.BlockSpec((1, 128), lambda g: (0, 0)),
            pl.BlockSpec((768, 512), lambda g: (0, 0)),
            pl.BlockSpec((1, 128), lambda g: (0, 0)),
            pl.BlockSpec((5, 128, 128), lambda g: (0, 0, 0)),
            pl.BlockSpec((1, 128), lambda g: (0, 0)),
            pl.BlockSpec((128, 128), lambda g: (0, 0)),
            pl.BlockSpec((1, 128), lambda g: (0, 0)),
            pl.BlockSpec((128, 128), lambda g: (0, 0)),
            pl.BlockSpec((1, 128), lambda g: (0, 0)),
        ],
        out_specs=pl.BlockSpec((bb, 128), lambda g: (g, 0)),
        compiler_params=pltpu.CompilerParams(
            dimension_semantics=("parallel",)),
    )(xro, w1b, b1p, w2b, b2p, f1w, fc1_b,
      fc2_w.astype(jnp.bfloat16), fc2_b, fc3_w.astype(jnp.bfloat16), fc3_b)
    return out[:, :2]


# speedup vs baseline: 1.2020x; 1.0140x over previous
import numpy as np
import jax
import jax.numpy as jnp
from jax.experimental import pallas as pl
from jax.experimental.pallas import tpu as pltpu


def _band_sel(k, cin, cout, n_px, lane_fn):
    rows = k * k * cin
    sel = np.zeros((6, 128, 4, n_px, rows + 1), np.float32)
    sel[..., rows] = 1.0
    for qy in range(2):
        for qx in range(2):
            q = qy * 2 + qx
            for dy in range(k):
                for dx in range(k):
                    for ci in range(cin):
                        for px in range(n_px):
                            i = qy + dy
                            l = lane_fn(px, qx + dx, ci)
                            r = dy * k * cin + dx * cin + ci
                            sel[i, l, q, px, rows] = 0.0
                            sel[i, l, q, px, r] = 1.0
    return sel.reshape(-1, rows + 1)


_W1_SEL = _band_sel(5, 3, 6, 14, lambda px, d, ci: 32 * ci + 2 * px + d)
_W2_SEL = _band_sel(5, 6, 16, 5, lambda px, d, ci: 6 * (2 * px + d) + ci)


def _banded(w_flat, sel, n_px, cout):
    cols = jnp.concatenate([w_flat[:, :cout],
                            jnp.zeros((1, cout), w_flat.dtype)]).astype(jnp.bfloat16)
    band = jnp.dot(sel, cols, preferred_element_type=jnp.float32)
    band = band.reshape(6, 128, 4, n_px * cout)
    band = jnp.pad(band, ((0, 0), (0, 0), (0, 0), (0, 128 - n_px * cout)))
    return band.reshape(768, 512).astype(jnp.bfloat16)


def _pool_bias_relu(h, b):
    m = jnp.maximum(jnp.maximum(h[:, 0:128], h[:, 128:256]),
                    jnp.maximum(h[:, 256:384], h[:, 384:512]))
    return jnp.maximum(m + b, 0.0)


def _net_kernel(x_ref, w1_ref, b1_ref, w2_ref, b2_ref,
                f1_ref, f1b_ref, f2_ref, f2b_ref, f3_ref, f3b_ref, o_ref):
    bb = x_ref.shape[1]
    x = x_ref[...]

    starts = [(16 if i % 2 else 0) + i // 2 for i in range(6)]
    a = jnp.concatenate([x[s:s + 14] for s in starts], axis=2)
    h = jnp.dot(a.reshape(14 * bb, 768), w1_ref[...],
                preferred_element_type=jnp.float32)
    y1 = _pool_bias_relu(h, b1_ref[...]).astype(jnp.bfloat16).reshape(14, bb, 128)
    y1 = jnp.concatenate([y1[2 * p:2 * p + 1] for p in range(7)] +
                         [y1[2 * p + 1:2 * p + 2] for p in range(7)], axis=0)

    starts = [(7 if i % 2 else 0) + i // 2 for i in range(6)]
    a = jnp.concatenate([y1[s:s + 5] for s in starts], axis=2)
    h = jnp.dot(a.reshape(5 * bb, 768), w2_ref[...],
                preferred_element_type=jnp.float32)
    y2 = _pool_bias_relu(h, b2_ref[...]).astype(jnp.bfloat16).reshape(5, bb, 128)

    h = None
    for r in range(5):
        d = jnp.dot(y2[r], f1_ref[r], preferred_element_type=jnp.float32)
        h = d if h is None else h + d
    h = jnp.maximum(h + f1b_ref[...], 0.0).astype(jnp.bfloat16)
    h = jnp.dot(h, f2_ref[...], preferred_element_type=jnp.float32)
    h = jnp.maximum(h + f2b_ref[...], 0.0).astype(jnp.bfloat16)
    h = jnp.dot(h, f3_ref[...], preferred_element_type=jnp.float32)
    o_ref[...] = h + f3b_ref[...]


_ROW_PERM = np.concatenate([np.arange(0, 32, 2), np.arange(1, 32, 2)])


def kernel(x, conv1_w, conv1_b, conv2_w, conv2_b, fc1_w, fc1_b,
           fc2_w, fc2_b, fc3_w, fc3_b):
    B = x.shape[0]
    bb = 512 if B % 512 == 0 else B
    grid = B // bb

    xro = jnp.transpose(x, (2, 0, 1, 3))[_ROW_PERM].reshape(32, B, 96)
    xro = jnp.pad(xro, ((0, 0), (0, 0), (0, 32))).astype(jnp.bfloat16)

    w1b = _banded(conv1_w, jnp.asarray(_W1_SEL, jnp.bfloat16), 14, 6)
    w2b = _banded(conv2_w, jnp.asarray(_W2_SEL, jnp.bfloat16), 5, 16)
    b1p = jnp.pad(jnp.tile(conv1_b[:, :6], (1, 16)), ((0, 0), (0, 32)))
    b2p = jnp.tile(conv2_b[:, :16], (1, 8))
    f1w = jnp.pad(fc1_w[:400].reshape(5, 80, 128),
                  ((0, 0), (0, 48), (0, 0))).astype(jnp.bfloat16)

    out = pl.pallas_call(
        _net_kernel,
        grid=(grid,),
        out_shape=jax.ShapeDtypeStruct((B, 128), jnp.float32),
        in_specs=[
            pl.BlockSpec((32, bb, 128), lambda g: (0, g, 0)),
            pl.BlockSpec((768, 512), lambda g: (0, 0)),
            pl.BlockSpec((1, 128), lambda g: (0, 0)),
            pl.BlockSpec((768, 512), lambda g: (0, 0)),
            pl.BlockSpec((1, 128), lambda g: (0, 0)),
            pl.BlockSpec((5, 128, 128), lambda g: (0, 0, 0)),
            pl.BlockSpec((1, 128), lambda g: (0, 0)),
            pl.BlockSpec((128, 128), lambda g: (0, 0)),
            pl.BlockSpec((1, 128), lambda g: (0, 0)),
            pl.BlockSpec((128, 128), lambda g: (0, 0)),
            pl.BlockSpec((1, 128), lambda g: (0, 0)),
        ],
        out_specs=pl.BlockSpec((bb, 128), lambda g: (g, 0)),
        compiler_params=pltpu.CompilerParams(
            dimension_semantics=("parallel",)),
    )(xro, w1b, b1p, w2b, b2p, f1w, fc1_b,
      fc2_w.astype(jnp.bfloat16), fc2_b, fc3_w.astype(jnp.bfloat16), fc3_b)
    return out[:, :2]
